# Initial kernel scaffold; baseline (speedup 1.0000x reference)
import jax
import jax.numpy as jnp
from jax.experimental import pallas as pl


def kernel(idx, target, token_embed):
    raise NotImplementedError("write your pallas kernel here")


# speedup vs baseline: 1.1555x; 1.1555x over previous
import functools

import jax
import jax.numpy as jnp
from jax import lax
from jax.experimental import pallas as pl
from jax.experimental.pallas import tpu as pltpu
from jax.experimental.pallas import tpu_sc as plsc

VOCAB_N = 1000
PAD_V = 1024
LANES = 16
CHUNK = 32
EG = 128


def _logz_tc(table_padded):
    def body(x_ref, o_ref):
        x = x_ref[...]
        m = jnp.max(x, axis=1)
        s = jnp.sum(jnp.exp(x - m[:, None]), axis=1)
        o_ref[...] = m + jnp.log(s)

    return pl.pallas_call(
        body,
        out_shape=jax.ShapeDtypeStruct((PAD_V,), jnp.float32),
    )(table_padded)


@functools.partial(jax.jit, static_argnums=(4,))
def _sc_gather_loss(idx_flat, comb_flat, table, aux, total_rows):
    info = plsc.get_sparse_core_info()
    nw = info.num_cores * info.num_subcores
    rows_per_w = total_rows // nw
    chunks = rows_per_w // CHUNK
    ne = rows_per_w // EG
    mesh = plsc.VectorSubcoreMesh(core_axis_name="c", subcore_axis_name="s")

    @functools.partial(
        pl.kernel,
        mesh=mesh,
        compiler_params=pltpu.CompilerParams(use_tc_tiling_on_sc=False),
        out_type=[
            jax.ShapeDtypeStruct((total_rows, VOCAB_N), jnp.float32),
            jax.ShapeDtypeStruct((nw, LANES), jnp.float32),
        ],
        scratch_types=[
            pltpu.VMEM((rows_per_w,), jnp.int32),
            pltpu.VMEM((rows_per_w,), jnp.int32),
            pltpu.VMEM((rows_per_w,), jnp.float32),
            pltpu.VMEM((rows_per_w,), jnp.float32),
            pltpu.VMEM((CHUNK, VOCAB_N), jnp.float32),
            pltpu.VMEM((CHUNK, VOCAB_N), jnp.float32),
            pltpu.VMEM((LANES,), jnp.float32),
            pltpu.SemaphoreType.DMA,
            pltpu.SemaphoreType.DMA,
            pltpu.SemaphoreType.DMA,
        ],
    )
    def k(idx_hbm, comb_hbm, table_hbm, aux_hbm, out_hbm, part_hbm,
          idx_v, comb_v, lz_v, pk_v, rows_a, rows_b, acc_v,
          gsem_a, gsem_b, esem):
        wid = lax.axis_index("s") * info.num_cores + lax.axis_index("c")
        base = wid * rows_per_w
        pltpu.sync_copy(idx_hbm.at[pl.ds(base, rows_per_w)], idx_v)
        pltpu.sync_copy(comb_hbm.at[pl.ds(base, rows_per_w)], comb_v)

        def fire(e, c):
            s = e * EG
            pltpu.async_copy(
                aux_hbm.at[idx_v.at[pl.ds(s, EG)]], lz_v.at[pl.ds(s, EG)],
                esem)
            pltpu.async_copy(
                aux_hbm.at[comb_v.at[pl.ds(s, EG)]], pk_v.at[pl.ds(s, EG)],
                esem)
            return c

        lax.fori_loop(0, ne, fire, 0)

        def start_gather(g, rows, sem):
            return pltpu.async_copy(
                table_hbm.at[idx_v.at[pl.ds(g * CHUNK, CHUNK)]], rows, sem)

        start_gather(0, rows_a, gsem_a).wait()

        def step(h, c):
            for (g_off, rows, o_rows, o_sem) in (
                (0, rows_a, rows_b, gsem_b),
                (1, rows_b, rows_a, gsem_a),
            ):
                g = h * 2 + g_off

                @pl.when(g + 1 < chunks)
                def _():
                    start_gather(g + 1, o_rows, o_sem)

                pltpu.sync_copy(rows, out_hbm.at[pl.ds(base + g * CHUNK, CHUNK)])

                @pl.when(g + 1 < chunks)
                def _():
                    pltpu.make_async_copy(
                        table_hbm.at[idx_v.at[pl.ds(0, CHUNK)]], o_rows, o_sem
                    ).wait()

            return c

        lax.fori_loop(0, chunks // 2, step, 0)

        def drain(e, c):
            s = e * EG
            pltpu.make_async_copy(
                aux_hbm.at[idx_v.at[pl.ds(s, EG)]], lz_v.at[pl.ds(s, EG)],
                esem).wait()
            pltpu.make_async_copy(
                aux_hbm.at[comb_v.at[pl.ds(s, EG)]], pk_v.at[pl.ds(s, EG)],
                esem).wait()
            return c

        lax.fori_loop(0, ne, drain, 0)

        def red(i, acc):
            s = i * LANES
            return acc + (lz_v[pl.ds(s, LANES)] - pk_v[pl.ds(s, LANES)])

        acc = lax.fori_loop(0, rows_per_w // LANES, red,
                            jnp.zeros((LANES,), jnp.float32))
        acc_v[...] = acc
        pltpu.sync_copy(acc_v, part_hbm.at[wid])

    return k(idx_flat, comb_flat, table, aux)


def kernel(idx, target, token_embed):
    B, T = idx.shape
    total = B * T
    idx_flat = idx.reshape(total).astype(jnp.int32)
    tgt_flat = target.reshape(total).astype(jnp.int32)
    comb_flat = PAD_V + idx_flat * VOCAB_N + tgt_flat
    table = token_embed.astype(jnp.float32)
    table_padded = jnp.zeros((PAD_V, VOCAB_N), jnp.float32).at[:VOCAB_N].set(table)
    logz = _logz_tc(table_padded)
    aux = jnp.concatenate([logz, table.reshape(VOCAB_N * VOCAB_N)])
    out, part = _sc_gather_loss(idx_flat, comb_flat, table, aux, total)
    loss = jnp.sum(part) / total
    return (out, loss)


# speedup vs baseline: 1.1618x; 1.0055x over previous
import functools

import jax
import jax.numpy as jnp
from jax import lax
from jax.experimental import pallas as pl
from jax.experimental.pallas import tpu as pltpu
from jax.experimental.pallas import tpu_sc as plsc

VOCAB_N = 1000
PAD_V = 1024
LANES = 16
CHUNK = 16
NBUF = 4
EG = 128


def _logz_tc(table_padded):
    def body(x_ref, o_ref):
        x = x_ref[...]
        m = jnp.max(x, axis=1)
        s = jnp.sum(jnp.exp(x - m[:, None]), axis=1)
        o_ref[...] = m + jnp.log(s)

    return pl.pallas_call(
        body,
        out_shape=jax.ShapeDtypeStruct((PAD_V,), jnp.float32),
    )(table_padded)


@functools.partial(jax.jit, static_argnums=(4,))
def _sc_gather_loss(idx_flat, comb_flat, table, aux, total_rows):
    info = plsc.get_sparse_core_info()
    nw = info.num_cores * info.num_subcores
    rows_per_w = total_rows // nw
    chunks = rows_per_w // CHUNK
    groups = chunks // NBUF
    ne = rows_per_w // EG
    mesh = plsc.VectorSubcoreMesh(core_axis_name="c", subcore_axis_name="s")

    @functools.partial(
        pl.kernel,
        mesh=mesh,
        compiler_params=pltpu.CompilerParams(use_tc_tiling_on_sc=False),
        out_type=[
            jax.ShapeDtypeStruct((total_rows, VOCAB_N), jnp.float32),
            jax.ShapeDtypeStruct((nw, LANES), jnp.float32),
        ],
        scratch_types=[
            pltpu.VMEM((rows_per_w,), jnp.int32),
            pltpu.VMEM((rows_per_w,), jnp.int32),
            pltpu.VMEM((rows_per_w,), jnp.float32),
            pltpu.VMEM((rows_per_w,), jnp.float32),
            [pltpu.VMEM((CHUNK, VOCAB_N), jnp.float32) for _ in range(NBUF)],
            pltpu.VMEM((LANES,), jnp.float32),
            [pltpu.SemaphoreType.DMA for _ in range(NBUF)],
            [pltpu.SemaphoreType.DMA for _ in range(NBUF)],
            pltpu.SemaphoreType.DMA,
        ],
    )
    def k(idx_hbm, comb_hbm, table_hbm, aux_hbm, out_hbm, part_hbm,
          idx_v, comb_v, lz_v, pk_v, rows, acc_v, gs, ws, esem):
        wid = lax.axis_index("s") * info.num_cores + lax.axis_index("c")
        base = wid * rows_per_w
        pltpu.sync_copy(idx_hbm.at[pl.ds(base, rows_per_w)], idx_v)
        pltpu.sync_copy(comb_hbm.at[pl.ds(base, rows_per_w)], comb_v)

        def fire(e, c):
            s = e * EG
            pltpu.async_copy(
                aux_hbm.at[idx_v.at[pl.ds(s, EG)]], lz_v.at[pl.ds(s, EG)],
                esem)
            pltpu.async_copy(
                aux_hbm.at[comb_v.at[pl.ds(s, EG)]], pk_v.at[pl.ds(s, EG)],
                esem)
            return c

        lax.fori_loop(0, ne, fire, 0)

        def start_gather(g, b):
            pltpu.async_copy(
                table_hbm.at[idx_v.at[pl.ds(g * CHUNK, CHUNK)]], rows[b],
                gs[b])

        def wait_gather(b):
            pltpu.make_async_copy(
                table_hbm.at[idx_v.at[pl.ds(0, CHUNK)]], rows[b], gs[b]
            ).wait()

        def start_write(g, b):
            pltpu.async_copy(
                rows[b], out_hbm.at[pl.ds(base + g * CHUNK, CHUNK)], ws[b])

        def wait_write(b):
            pltpu.make_async_copy(
                rows[b], out_hbm.at[pl.ds(0, CHUNK)], ws[b]).wait()

        D = NBUF - 1
        for b in range(D):
            start_gather(b, b)

        for b in range(NBUF):
            wait_gather(b)
            start_write(b, b)
            nb = (b + D) % NBUF
            if b + D >= NBUF:
                wait_write(nb)
            start_gather(b + D, nb)

        def step(h, c):
            for b in range(NBUF):
                g = h * NBUF + b
                wait_gather(b)
                start_write(g, b)
                nb = (b + D) % NBUF
                wait_write(nb)
                start_gather(g + D, nb)
            return c

        lax.fori_loop(1, groups - 1, step, 0)

        for b in range(NBUF):
            g = (groups - 1) * NBUF + b
            wait_gather(b)
            start_write(g, b)
            if g + D < chunks:
                nb = (b + D) % NBUF
                wait_write(nb)
                start_gather(g + D, nb)
        for b in range(NBUF):
            wait_write(b)

        def drain(e, c):
            s = e * EG
            pltpu.make_async_copy(
                aux_hbm.at[idx_v.at[pl.ds(s, EG)]], lz_v.at[pl.ds(s, EG)],
                esem).wait()
            pltpu.make_async_copy(
                aux_hbm.at[comb_v.at[pl.ds(s, EG)]], pk_v.at[pl.ds(s, EG)],
                esem).wait()
            return c

        lax.fori_loop(0, ne, drain, 0)

        def red(i, acc):
            s = i * LANES
            return acc + (lz_v[pl.ds(s, LANES)] - pk_v[pl.ds(s, LANES)])

        acc = lax.fori_loop(0, rows_per_w // LANES, red,
                            jnp.zeros((LANES,), jnp.float32))
        acc_v[...] = acc
        pltpu.sync_copy(acc_v, part_hbm.at[wid])

    return k(idx_flat, comb_flat, table, aux)


def kernel(idx, target, token_embed):
    B, T = idx.shape
    total = B * T
    idx_flat = idx.reshape(total).astype(jnp.int32)
    tgt_flat = target.reshape(total).astype(jnp.int32)
    comb_flat = PAD_V + idx_flat * VOCAB_N + tgt_flat
    table = token_embed.astype(jnp.float32)
    table_padded = jnp.zeros((PAD_V, VOCAB_N), jnp.float32).at[:VOCAB_N].set(table)
    logz = _logz_tc(table_padded)
    aux = jnp.concatenate([logz, table.reshape(VOCAB_N * VOCAB_N)])
    out, part = _sc_gather_loss(idx_flat, comb_flat, table, aux, total)
    loss = jnp.sum(part) / total
    return (out, loss)


# speedup vs baseline: 1.9134x; 1.6469x over previous
import functools

import jax
import jax.numpy as jnp
from jax import lax
from jax.experimental import pallas as pl
from jax.experimental.pallas import tpu as pltpu
from jax.experimental.pallas import tpu_sc as plsc

VOCAB_N = 1000
PAD_V = 1024
LANES = 16
CHUNK = 16
NBUF = 4
EG = 128


def _logz_tc(table_padded):
    def body(x_ref, o_ref):
        x = x_ref[...]
        m = jnp.max(x, axis=1)
        s = jnp.sum(jnp.exp(x - m[:, None]), axis=1)
        o_ref[...] = m + jnp.log(s)

    return pl.pallas_call(
        body,
        out_shape=jax.ShapeDtypeStruct((PAD_V,), jnp.float32),
    )(table_padded)


@functools.partial(jax.jit, static_argnums=(4,))
def _sc_gather_loss(idx_flat, comb_flat, table, aux, total_rows):
    info = plsc.get_sparse_core_info()
    nw = info.num_cores * info.num_subcores
    rows_per_w = total_rows // nw
    chunks = rows_per_w // CHUNK
    groups = chunks // NBUF
    ne = rows_per_w // EG
    mesh = plsc.VectorSubcoreMesh(core_axis_name="c", subcore_axis_name="s")

    @functools.partial(
        pl.kernel,
        mesh=mesh,
        compiler_params=pltpu.CompilerParams(use_tc_tiling_on_sc=True),
        out_type=[
            jax.ShapeDtypeStruct((total_rows, PAD_V), jnp.float32),
            jax.ShapeDtypeStruct((nw * LANES,), jnp.float32),
        ],
        scratch_types=[
            pltpu.VMEM((rows_per_w,), jnp.int32),
            pltpu.VMEM((rows_per_w,), jnp.int32),
            pltpu.VMEM((rows_per_w,), jnp.float32),
            pltpu.VMEM((rows_per_w,), jnp.float32),
            [pltpu.VMEM((CHUNK, PAD_V), jnp.float32) for _ in range(NBUF)],
            pltpu.VMEM((LANES,), jnp.float32),
            [pltpu.SemaphoreType.DMA for _ in range(NBUF)],
            [pltpu.SemaphoreType.DMA for _ in range(NBUF)],
            pltpu.SemaphoreType.DMA,
        ],
    )
    def k(idx_hbm, comb_hbm, table_hbm, aux_hbm, out_hbm, part_hbm,
          idx_v, comb_v, lz_v, pk_v, rows, acc_v, gs, ws, esem):
        wid = lax.axis_index("s") * info.num_cores + lax.axis_index("c")
        base = wid * rows_per_w
        pltpu.sync_copy(idx_hbm.at[pl.ds(base, rows_per_w)], idx_v)
        pltpu.sync_copy(comb_hbm.at[pl.ds(base, rows_per_w)], comb_v)

        def fire(e, c):
            s = e * EG
            pltpu.async_copy(
                aux_hbm.at[idx_v.at[pl.ds(s, EG)]], lz_v.at[pl.ds(s, EG)],
                esem)
            pltpu.async_copy(
                aux_hbm.at[comb_v.at[pl.ds(s, EG)]], pk_v.at[pl.ds(s, EG)],
                esem)
            return c

        lax.fori_loop(0, ne, fire, 0)

        def start_gather(g, b):
            pltpu.async_copy(
                table_hbm.at[idx_v.at[pl.ds(g * CHUNK, CHUNK)]], rows[b],
                gs[b])

        def wait_gather(b):
            pltpu.make_async_copy(
                table_hbm.at[idx_v.at[pl.ds(0, CHUNK)]], rows[b], gs[b]
            ).wait()

        def start_write(g, b):
            pltpu.async_copy(
                rows[b], out_hbm.at[pl.ds(base + g * CHUNK, CHUNK)], ws[b])

        def wait_write(b):
            pltpu.make_async_copy(
                rows[b], out_hbm.at[pl.ds(0, CHUNK)], ws[b]).wait()

        D = NBUF - 1
        for b in range(D):
            start_gather(b, b)

        for b in range(NBUF):
            wait_gather(b)
            start_write(b, b)
            nb = (b + D) % NBUF
            if b + D >= NBUF:
                wait_write(nb)
            start_gather(b + D, nb)

        def step(h, c):
            for b in range(NBUF):
                g = h * NBUF + b
                wait_gather(b)
                start_write(g, b)
                nb = (b + D) % NBUF
                wait_write(nb)
                start_gather(g + D, nb)
            return c

        lax.fori_loop(1, groups - 1, step, 0)

        for b in range(NBUF):
            g = (groups - 1) * NBUF + b
            wait_gather(b)
            start_write(g, b)
            if g + D < chunks:
                nb = (b + D) % NBUF
                wait_write(nb)
                start_gather(g + D, nb)
        for b in range(NBUF):
            wait_write(b)

        def drain(e, c):
            s = e * EG
            pltpu.make_async_copy(
                aux_hbm.at[idx_v.at[pl.ds(s, EG)]], lz_v.at[pl.ds(s, EG)],
                esem).wait()
            pltpu.make_async_copy(
                aux_hbm.at[comb_v.at[pl.ds(s, EG)]], pk_v.at[pl.ds(s, EG)],
                esem).wait()
            return c

        lax.fori_loop(0, ne, drain, 0)

        def red(i, acc):
            s = i * LANES
            return acc + (lz_v[pl.ds(s, LANES)] - pk_v[pl.ds(s, LANES)])

        acc = lax.fori_loop(0, rows_per_w // LANES, red,
                            jnp.zeros((LANES,), jnp.float32))
        acc_v[...] = acc
        pltpu.sync_copy(acc_v, part_hbm.at[pl.ds(wid * LANES, LANES)])

    return k(idx_flat, comb_flat, table, aux)


def kernel(idx, target, token_embed):
    B, T = idx.shape
    total = B * T
    idx_flat = idx.reshape(total).astype(jnp.int32)
    tgt_flat = target.reshape(total).astype(jnp.int32)
    comb_flat = PAD_V + idx_flat * VOCAB_N + tgt_flat
    table = token_embed.astype(jnp.float32)
    table_padded = jnp.zeros((PAD_V, VOCAB_N), jnp.float32).at[:VOCAB_N].set(table)
    table_cols = jnp.zeros((VOCAB_N, PAD_V), jnp.float32).at[:, :VOCAB_N].set(table)
    logz = _logz_tc(table_padded)
    aux = jnp.concatenate([logz, table.reshape(VOCAB_N * VOCAB_N)])
    out, part = _sc_gather_loss(idx_flat, comb_flat, table_cols, aux, total)
    loss = jnp.sum(part) / total
    return (out[:, :VOCAB_N], loss)


# speedup vs baseline: 1.9134x; 1.0000x over previous
import functools

import jax
import jax.numpy as jnp
from jax import lax
from jax.experimental import pallas as pl
from jax.experimental.pallas import tpu as pltpu
from jax.experimental.pallas import tpu_sc as plsc

VOCAB_N = 1000
PAD_V = 1024
LANES = 16
CHUNK = 16
NBUF = 5
EG = 128


def _logz_tc(table_padded):
    def body(x_ref, o_ref):
        x = x_ref[...]
        m = jnp.max(x, axis=1)
        s = jnp.sum(jnp.exp(x - m[:, None]), axis=1)
        o_ref[...] = m + jnp.log(s)

    return pl.pallas_call(
        body,
        out_shape=jax.ShapeDtypeStruct((PAD_V,), jnp.float32),
    )(table_padded)


@functools.partial(jax.jit, static_argnums=(4,))
def _sc_gather_loss(idx_flat, comb_flat, table, aux, total_rows):
    info = plsc.get_sparse_core_info()
    nw = info.num_cores * info.num_subcores
    rows_per_w = total_rows // nw
    chunks = rows_per_w // CHUNK
    groups = chunks // NBUF
    ne = rows_per_w // EG
    mesh = plsc.VectorSubcoreMesh(core_axis_name="c", subcore_axis_name="s")

    @functools.partial(
        pl.kernel,
        mesh=mesh,
        compiler_params=pltpu.CompilerParams(use_tc_tiling_on_sc=True),
        out_type=[
            jax.ShapeDtypeStruct((total_rows, PAD_V), jnp.float32),
            jax.ShapeDtypeStruct((nw * LANES,), jnp.float32),
        ],
        scratch_types=[
            pltpu.VMEM((rows_per_w,), jnp.int32),
            pltpu.VMEM((rows_per_w,), jnp.int32),
            pltpu.VMEM((rows_per_w,), jnp.float32),
            pltpu.VMEM((rows_per_w,), jnp.float32),
            [pltpu.VMEM((CHUNK, PAD_V), jnp.float32) for _ in range(NBUF)],
            pltpu.VMEM((LANES,), jnp.float32),
            [pltpu.SemaphoreType.DMA for _ in range(NBUF)],
            [pltpu.SemaphoreType.DMA for _ in range(NBUF)],
            pltpu.SemaphoreType.DMA,
        ],
    )
    def k(idx_hbm, comb_hbm, table_hbm, aux_hbm, out_hbm, part_hbm,
          idx_v, comb_v, lz_v, pk_v, rows, acc_v, gs, ws, esem):
        wid = lax.axis_index("s") * info.num_cores + lax.axis_index("c")
        base = wid * rows_per_w
        pltpu.sync_copy(idx_hbm.at[pl.ds(base, rows_per_w)], idx_v)
        pltpu.sync_copy(comb_hbm.at[pl.ds(base, rows_per_w)], comb_v)

        def fire(e, c):
            s = e * EG
            pltpu.async_copy(
                aux_hbm.at[idx_v.at[pl.ds(s, EG)]], lz_v.at[pl.ds(s, EG)],
                esem)
            pltpu.async_copy(
                aux_hbm.at[comb_v.at[pl.ds(s, EG)]], pk_v.at[pl.ds(s, EG)],
                esem)
            return c

        lax.fori_loop(0, ne, fire, 0)

        def start_gather(g, b):
            pltpu.async_copy(
                table_hbm.at[idx_v.at[pl.ds(g * CHUNK, CHUNK)]], rows[b],
                gs[b])

        def wait_gather(b):
            pltpu.make_async_copy(
                table_hbm.at[idx_v.at[pl.ds(0, CHUNK)]], rows[b], gs[b]
            ).wait()

        def start_write(g, b):
            pltpu.async_copy(
                rows[b], out_hbm.at[pl.ds(base + g * CHUNK, CHUNK)], ws[b])

        def wait_write(b):
            pltpu.make_async_copy(
                rows[b], out_hbm.at[pl.ds(0, CHUNK)], ws[b]).wait()

        D = NBUF - 1
        for b in range(D):
            start_gather(b, b)

        for b in range(NBUF):
            wait_gather(b)
            start_write(b, b)
            nb = (b + D) % NBUF
            if b + D >= NBUF:
                wait_write(nb)
            start_gather(b + D, nb)

        def step(h, c):
            for b in range(NBUF):
                g = h * NBUF + b
                wait_gather(b)
                start_write(g, b)
                nb = (b + D) % NBUF
                wait_write(nb)
                start_gather(g + D, nb)
            return c

        lax.fori_loop(1, groups - 1, step, 0)

        for b in range(NBUF):
            g = (groups - 1) * NBUF + b
            wait_gather(b)
            start_write(g, b)
            if g + D < chunks:
                nb = (b + D) % NBUF
                wait_write(nb)
                start_gather(g + D, nb)
        for b in range(NBUF):
            wait_write(b)

        def drain(e, c):
            s = e * EG
            pltpu.make_async_copy(
                aux_hbm.at[idx_v.at[pl.ds(s, EG)]], lz_v.at[pl.ds(s, EG)],
                esem).wait()
            pltpu.make_async_copy(
                aux_hbm.at[comb_v.at[pl.ds(s, EG)]], pk_v.at[pl.ds(s, EG)],
                esem).wait()
            return c

        lax.fori_loop(0, ne, drain, 0)

        def red(i, acc):
            s = i * LANES
            return acc + (lz_v[pl.ds(s, LANES)] - pk_v[pl.ds(s, LANES)])

        acc = lax.fori_loop(0, rows_per_w // LANES, red,
                            jnp.zeros((LANES,), jnp.float32))
        acc_v[...] = acc
        pltpu.sync_copy(acc_v, part_hbm.at[pl.ds(wid * LANES, LANES)])

    return k(idx_flat, comb_flat, table, aux)


def kernel(idx, target, token_embed):
    B, T = idx.shape
    total = B * T
    idx_flat = idx.reshape(total).astype(jnp.int32)
    tgt_flat = target.reshape(total).astype(jnp.int32)
    comb_flat = PAD_V + idx_flat * VOCAB_N + tgt_flat
    table = token_embed.astype(jnp.float32)
    table_padded = jnp.zeros((PAD_V, VOCAB_N), jnp.float32).at[:VOCAB_N].set(table)
    table_cols = jnp.zeros((VOCAB_N, PAD_V), jnp.float32).at[:, :VOCAB_N].set(table)
    logz = _logz_tc(table_padded)
    aux = jnp.concatenate([logz, table.reshape(VOCAB_N * VOCAB_N)])
    out, part = _sc_gather_loss(idx_flat, comb_flat, table_cols, aux, total)
    loss = jnp.sum(part) / total
    return (out[:, :VOCAB_N], loss)
